# baseline (speedup 1.0000x reference)
import jax
import jax.numpy as jnp
from jax import lax
from jax.experimental import pallas as pl
from jax.experimental.pallas import tpu as pltpu
from jax.experimental.pallas import tpu_sc as plsc

NUM_ENT = 100000
HDIM = 128
NUM_REL = 16
B = 16384

NC, NS, L = 2, 16, 16
NW = NC * NS
CHUNK = B // NW
SUB = 128
NSUB = CHUNK // SUB
NBLK = HDIM // L

_DNUMS = lax.GatherDimensionNumbers(
    offset_dims=(), collapsed_slice_dims=(0,), start_index_map=(0,))


def _splat(vec, idx):
    return lax.gather(vec, idx[:, None], _DNUMS, (1,),
                      mode=lax.GatherScatterMode.PROMISE_IN_BOUNDS)


def _body(x_hbm, y_hbm, r_hbm, tab_hbm, R_hbm, out_hbm,
          xi0, xi1, yi0, yi1, rv, Rv, xr0, xr1, yr0, yr1, sc,
          sx0a, sx0b, sx1a, sx1b, sy0a, sy0b, sy1a, sy1b,
          six0, six1, siy0, siy1, srv, sRv):
    wid = lax.axis_index("s") * NC + lax.axis_index("c")
    base = wid * CHUNK
    xis, yis = [xi0, xi1], [yi0, yi1]
    xrs, yrs = [xr0, xr1], [yr0, yr1]
    sxs = [[sx0a, sx0b], [sx1a, sx1b]]
    sys_ = [[sy0a, sy0b], [sy1a, sy1b]]
    sixs, siys = [six0, six1], [siy0, siy1]
    H = SUB // 2

    crv = pltpu.async_copy(r_hbm.at[pl.ds(base, CHUNK)], rv, srv)
    cRv = pltpu.async_copy(R_hbm, Rv, sRv)

    def start(sub):
        k = sub % 2
        off = base + sub * SUB
        cix = pltpu.async_copy(x_hbm.at[pl.ds(off, SUB)], xis[k], sixs[k])
        ciy = pltpu.async_copy(y_hbm.at[pl.ds(off, SUB)], yis[k], siys[k])
        cix.wait()
        cxa = pltpu.async_copy(tab_hbm.at[xis[k].at[pl.ds(0, H)]],
                               xrs[k].at[pl.ds(0, H), :], sxs[k][0])
        cxb = pltpu.async_copy(tab_hbm.at[xis[k].at[pl.ds(H, H)]],
                               xrs[k].at[pl.ds(H, H), :], sxs[k][1])
        ciy.wait()
        cya = pltpu.async_copy(tab_hbm.at[yis[k].at[pl.ds(0, H)]],
                               yrs[k].at[pl.ds(0, H), :], sys_[k][0])
        cyb = pltpu.async_copy(tab_hbm.at[yis[k].at[pl.ds(H, H)]],
                               yrs[k].at[pl.ds(H, H), :], sys_[k][1])
        return cxa, cya, cxb, cyb

    lane = lax.broadcasted_iota(jnp.int32, (L,), 0)
    last = jnp.full((L,), L - 1, jnp.int32)
    pend = start(0)
    crv.wait()
    cRv.wait()
    for sub in range(NSUB):
        k = sub % 2
        cxa, cya, cxb, cyb = pend
        if sub + 1 < NSUB:
            pend = start(sub + 1)
        cxa.wait()
        cya.wait()
        xr, yr = xrs[k], yrs[k]

        def gbody(g, _, xr=xr, yr=yr, sub=sub):
            goff = g * L
            rvec = rv[pl.ds(sub * SUB + goff, L)]

            def ebody(j, out, xr=xr, yr=yr):
                e = goff + j
                re = _splat(rvec, jnp.full((L,), j, jnp.int32))
                parts = [xr[e, pl.ds(blk * L, L)]
                         * yr[e, pl.ds(blk * L, L)]
                         * plsc.load_gather(Rv, [re, lane + blk * L])
                         for blk in range(NBLK)]
                while len(parts) > 1:
                    parts = [parts[i] + parts[i + 1]
                             for i in range(0, len(parts) - 1, 2)] + (
                                 [parts[-1]] if len(parts) % 2 else [])
                tot = _splat(jnp.cumsum(parts[0]), last)
                return jnp.where(lane == j, tot, out)

            out = lax.fori_loop(0, L, ebody, jnp.zeros((L,), jnp.float32),
                                unroll=2)
            sc[pl.ds(sub * SUB + goff, L)] = out
            return 0

        lax.fori_loop(0, H // L, gbody, 0)
        cxb.wait()
        cyb.wait()
        lax.fori_loop(H // L, SUB // L, gbody, 0)

    pltpu.sync_copy(sc, out_hbm.at[pl.ds(base, CHUNK)])


@jax.jit
def kernel(x, y, r, emb_table, R):
    mesh = plsc.VectorSubcoreMesh(core_axis_name="c", subcore_axis_name="s")
    return pl.kernel(
        _body,
        out_type=jax.ShapeDtypeStruct((B,), jnp.float32),
        mesh=mesh,
        compiler_params=pltpu.CompilerParams(needs_layout_passes=False),
        scratch_types=[
            pltpu.VMEM((SUB,), jnp.int32),
            pltpu.VMEM((SUB,), jnp.int32),
            pltpu.VMEM((SUB,), jnp.int32),
            pltpu.VMEM((SUB,), jnp.int32),
            pltpu.VMEM((CHUNK,), jnp.int32),
            pltpu.VMEM((NUM_REL, HDIM), jnp.float32),
            pltpu.VMEM((SUB, HDIM), jnp.float32),
            pltpu.VMEM((SUB, HDIM), jnp.float32),
            pltpu.VMEM((SUB, HDIM), jnp.float32),
            pltpu.VMEM((SUB, HDIM), jnp.float32),
            pltpu.VMEM((CHUNK,), jnp.float32),
        ] + [pltpu.SemaphoreType.DMA] * 14,
    )(x, y, r, emb_table, R)


# speedup vs baseline: 1.0399x; 1.0399x over previous
import jax
import jax.numpy as jnp
from jax import lax
from jax.experimental import pallas as pl
from jax.experimental.pallas import tpu as pltpu
from jax.experimental.pallas import tpu_sc as plsc

NUM_ENT = 100000
HDIM = 128
NUM_REL = 16
B = 16384

NC, NS, L = 2, 16, 16
NW = NC * NS
CHUNK = B // NW
SUB = 128
NSUB = CHUNK // SUB
NBLK = HDIM // L

_DNUMS = lax.GatherDimensionNumbers(
    offset_dims=(), collapsed_slice_dims=(0,), start_index_map=(0,))


def _splat(vec, idx):
    return lax.gather(vec, idx[:, None], _DNUMS, (1,),
                      mode=lax.GatherScatterMode.PROMISE_IN_BOUNDS)


def _body(x_hbm, y_hbm, r_hbm, tab_hbm, R_hbm, out_hbm,
          xi0, xi1, yi0, yi1, rv, Rv, xr0, xr1, yr0, yr1, sc,
          sx0, sx1, sy0, sy1, six0, six1, siy0, siy1, srv, sRv):
    wid = lax.axis_index("s") * NC + lax.axis_index("c")
    base = wid * CHUNK
    xis, yis = [xi0, xi1], [yi0, yi1]
    xrs, yrs = [xr0, xr1], [yr0, yr1]
    sxs, sys_ = [sx0, sx1], [sy0, sy1]
    sixs, siys = [six0, six1], [siy0, siy1]

    crv = pltpu.async_copy(r_hbm.at[pl.ds(base, CHUNK)], rv, srv)
    cRv = pltpu.async_copy(R_hbm, Rv, sRv)

    def start(sub):
        k = sub % 2
        off = base + sub * SUB
        cix = pltpu.async_copy(x_hbm.at[pl.ds(off, SUB)], xis[k], sixs[k])
        ciy = pltpu.async_copy(y_hbm.at[pl.ds(off, SUB)], yis[k], siys[k])
        cix.wait()
        cx = pltpu.async_copy(tab_hbm.at[xis[k]], xrs[k], sxs[k])
        ciy.wait()
        cy = pltpu.async_copy(tab_hbm.at[yis[k]], yrs[k], sys_[k])
        return cx, cy

    lane = lax.broadcasted_iota(jnp.int32, (L,), 0)
    last = jnp.full((L,), L - 1, jnp.int32)
    pend = start(0)
    crv.wait()
    cRv.wait()
    for sub in range(NSUB):
        k = sub % 2
        cx, cy = pend
        if sub + 1 < NSUB:
            pend = start(sub + 1)
        cx.wait()
        cy.wait()
        xr, yr = xrs[k], yrs[k]

        def gbody(g, _, xr=xr, yr=yr, sub=sub):
            goff = g * L
            rvec = rv[pl.ds(sub * SUB + goff, L)]

            def ebody(j, out, xr=xr, yr=yr):
                e = goff + j
                re = _splat(rvec, jnp.full((L,), j, jnp.int32))
                parts = [xr[e, pl.ds(blk * L, L)]
                         * yr[e, pl.ds(blk * L, L)]
                         * plsc.load_gather(Rv, [re, lane + blk * L])
                         for blk in range(NBLK)]
                while len(parts) > 1:
                    parts = [parts[i] + parts[i + 1]
                             for i in range(0, len(parts) - 1, 2)] + (
                                 [parts[-1]] if len(parts) % 2 else [])
                tot = _splat(jnp.cumsum(parts[0]), last)
                return jnp.where(lane == j, tot, out)

            out = lax.fori_loop(0, L, ebody, jnp.zeros((L,), jnp.float32),
                                unroll=2)
            sc[pl.ds(sub * SUB + goff, L)] = out
            return 0

        lax.fori_loop(0, SUB // L, gbody, 0)

    pltpu.sync_copy(sc, out_hbm.at[pl.ds(base, CHUNK)])


@jax.jit
def kernel(x, y, r, emb_table, R):
    mesh = plsc.VectorSubcoreMesh(core_axis_name="c", subcore_axis_name="s")
    return pl.kernel(
        _body,
        out_type=jax.ShapeDtypeStruct((B,), jnp.float32),
        mesh=mesh,
        compiler_params=pltpu.CompilerParams(needs_layout_passes=False),
        scratch_types=[
            pltpu.VMEM((SUB,), jnp.int32),
            pltpu.VMEM((SUB,), jnp.int32),
            pltpu.VMEM((SUB,), jnp.int32),
            pltpu.VMEM((SUB,), jnp.int32),
            pltpu.VMEM((CHUNK,), jnp.int32),
            pltpu.VMEM((NUM_REL, HDIM), jnp.float32),
            pltpu.VMEM((SUB, HDIM), jnp.float32),
            pltpu.VMEM((SUB, HDIM), jnp.float32),
            pltpu.VMEM((SUB, HDIM), jnp.float32),
            pltpu.VMEM((SUB, HDIM), jnp.float32),
            pltpu.VMEM((CHUNK,), jnp.float32),
            pltpu.SemaphoreType.DMA,
            pltpu.SemaphoreType.DMA,
            pltpu.SemaphoreType.DMA,
            pltpu.SemaphoreType.DMA,
            pltpu.SemaphoreType.DMA,
            pltpu.SemaphoreType.DMA,
            pltpu.SemaphoreType.DMA,
            pltpu.SemaphoreType.DMA,
            pltpu.SemaphoreType.DMA,
            pltpu.SemaphoreType.DMA,
        ],
    )(x, y, r, emb_table, R)
